# baseline (speedup 1.0000x reference)
import functools

import jax
import jax.numpy as jnp
from jax import lax
from jax.experimental import pallas as pl
from jax.experimental.pallas import tpu as pltpu
from jax.experimental.pallas import tpu_sc as plsc

B, N = 4, 16384
NPOINT, RADIUS, NSAMPLE = 1024, 0.5, 32
R2 = RADIUS * RADIUS
BIG = 3.0e38
W = 80


def _fps_body(xyzT_ref, sidx_ref, nx0_ref, nx1_ref, nx2_ref, dists_ref):
    x0 = xyzT_ref[:, 0, :]
    x1 = xyzT_ref[:, 1, :]
    x2 = xyzT_ref[:, 2, :]
    dists_ref[...] = jnp.full((B, N), 1e10, jnp.float32)
    iota_n = lax.broadcasted_iota(jnp.int32, (B, N), 1)
    iota_m = lax.broadcasted_iota(jnp.int32, (B, NPOINT), 1)

    def body(i, far):
        sidx_ref[...] = jnp.where(iota_m == i, far, sidx_ref[...])
        onehot = (iota_n == far).astype(jnp.float32)
        c0 = jnp.sum(x0 * onehot, axis=1, keepdims=True)
        c1 = jnp.sum(x1 * onehot, axis=1, keepdims=True)
        c2 = jnp.sum(x2 * onehot, axis=1, keepdims=True)
        nx0_ref[...] = jnp.where(iota_m == i, c0, nx0_ref[...])
        nx1_ref[...] = jnp.where(iota_m == i, c1, nx1_ref[...])
        nx2_ref[...] = jnp.where(iota_m == i, c2, nx2_ref[...])
        d0 = x0 - c0
        d1 = x1 - c1
        d2 = x2 - c2
        d = d0 * d0 + d1 * d1 + d2 * d2
        nd = jnp.minimum(dists_ref[...], d)
        dists_ref[...] = nd
        m = jnp.max(nd, axis=1, keepdims=True)
        far_new = jnp.min(jnp.where(nd == m, iota_n, N), axis=1, keepdims=True)
        return far_new.astype(jnp.int32)

    lax.fori_loop(0, NPOINT, body, jnp.zeros((B, 1), jnp.int32))


def _fps(xyzT):
    return pl.pallas_call(
        _fps_body,
        grid=(1,),
        in_specs=[pl.BlockSpec((B, 3, N), lambda i: (0, 0, 0))],
        out_specs=[
            pl.BlockSpec((B, NPOINT), lambda i: (0, 0)),
            pl.BlockSpec((B, NPOINT), lambda i: (0, 0)),
            pl.BlockSpec((B, NPOINT), lambda i: (0, 0)),
            pl.BlockSpec((B, NPOINT), lambda i: (0, 0)),
        ],
        out_shape=[
            jax.ShapeDtypeStruct((B, NPOINT), jnp.int32),
            jax.ShapeDtypeStruct((B, NPOINT), jnp.float32),
            jax.ShapeDtypeStruct((B, NPOINT), jnp.float32),
            jax.ShapeDtypeStruct((B, NPOINT), jnp.float32),
        ],
        scratch_shapes=[pltpu.VMEM((B, N), jnp.float32)],
    )(xyzT)


MQ = 128
NC = 128
NL = 128
TOPC = 8


def _select_body(xyz3_ref, newxT_ref, out_ref, valid_ref):
    b = pl.program_id(0)
    bf = jnp.bfloat16
    x0 = xyz3_ref[0, 0]
    x1 = xyz3_ref[0, 1]
    x2 = xyz3_ref[0, 2]
    xx = (x0 * x0 + x1 * x1 + x2 * x2)[None]
    x0b = x0.astype(bf).astype(jnp.float32)[None]
    x1b = x1.astype(bf).astype(jnp.float32)[None]
    x2b = x2.astype(bf).astype(jnp.float32)[None]
    q0 = newxT_ref[0, 0, :].reshape(MQ, 1, 1)
    q1 = newxT_ref[0, 1, :].reshape(MQ, 1, 1)
    q2 = newxT_ref[0, 2, :].reshape(MQ, 1, 1)
    qq = q0 * q0 + q1 * q1 + q2 * q2
    q0b = q0.astype(bf).astype(jnp.float32)
    q1b = q1.astype(bf).astype(jnp.float32)
    q2b = q2.astype(bf).astype(jnp.float32)
    dot = q0b * x0b + q1b * x1b + q2b * x2b
    dist2 = (qq + xx) - 2.0 * dot
    D = jnp.where(dist2 <= R2, dist2, BIG)

    lane3 = lax.broadcasted_iota(jnp.int32, (MQ, NC, NL), 2)
    gbase = (lax.broadcasted_iota(jnp.int32, (MQ, NC), 1) * NL) + b * N
    vals, gidx = [], []
    for _ in range(TOPC):
        m_c = jnp.min(D, axis=2)
        lane_c = jnp.min(jnp.where(D == m_c[:, :, None], lane3, NL), axis=2)
        D = jnp.where(lane3 == lane_c[:, :, None], BIG, D)
        vals.append(m_c)
        gidx.append(gbase + lane_c)
    V = jnp.stack(vals, axis=1)
    G = jnp.stack(gidx, axis=1)

    valid_ref[0, 0] = (jnp.min(V[:, 0, :], axis=1) < BIG).astype(jnp.float32)
    iota_s = lax.broadcasted_iota(jnp.int32, (MQ, NSAMPLE), 1)
    IBIG = jnp.int32(2 ** 30)

    def body(s, state):
        V, first, acc = state
        m = jnp.min(jnp.min(V, axis=2, keepdims=True), axis=1, keepdims=True)
        hit = V == m
        idxv = jnp.min(jnp.min(jnp.where(hit, G, IBIG), axis=2, keepdims=True),
                       axis=1, keepdims=True)
        first = jnp.where(s == 0, idxv, first)
        chosen = jnp.where(m < BIG, idxv, first)
        acc = jnp.where(iota_s == s, chosen.reshape(MQ, 1), acc)
        V = jnp.where(G == idxv, BIG, V)
        return (V, first, acc)

    _, _, acc = lax.fori_loop(
        0, NSAMPLE, body,
        (V, jnp.zeros((MQ, 1, 1), jnp.int32), jnp.zeros((MQ, NSAMPLE), jnp.int32)))
    out_ref[0] = acc


def _select(xyz3, newxT):
    return pl.pallas_call(
        _select_body,
        grid=(B, NPOINT // MQ),
        in_specs=[
            pl.BlockSpec((1, 3, NC, NL), lambda b, i: (b, 0, 0, 0)),
            pl.BlockSpec((1, 3, MQ), lambda b, i: (b, 0, i)),
        ],
        out_specs=[
            pl.BlockSpec((1, MQ, NSAMPLE), lambda b, i: (b, i, 0)),
            pl.BlockSpec((1, 1, MQ), lambda b, i: (b, 0, i)),
        ],
        out_shape=[
            jax.ShapeDtypeStruct((B, NPOINT, NSAMPLE), jnp.int32),
            jax.ShapeDtypeStruct((B, 1, NPOINT), jnp.float32),
        ],
    )(xyz3, newxT)


NB = 2048


def _table_body(feat_ref, xyzT_ref, out_ref):
    eye = (lax.broadcasted_iota(jnp.int32, (64, 64), 0)
           == lax.broadcasted_iota(jnp.int32, (64, 64), 1)).astype(jnp.float32)
    ft = lax.dot_general(feat_ref[0], eye, (((0,), (0,)), ((), ())),
                         preferred_element_type=jnp.float32)
    out_ref[0, :, 0:64] = ft
    out_ref[0, :, 64:65] = xyzT_ref[0, 0, :].reshape(NB, 1)
    out_ref[0, :, 65:66] = xyzT_ref[0, 1, :].reshape(NB, 1)
    out_ref[0, :, 66:67] = xyzT_ref[0, 2, :].reshape(NB, 1)
    out_ref[0, :, 67:W] = jnp.zeros((NB, W - 67), jnp.float32)


def _build_table(features, xyzT):
    return pl.pallas_call(
        _table_body,
        grid=(B, N // NB),
        in_specs=[
            pl.BlockSpec((1, 64, NB), lambda b, i: (b, 0, i)),
            pl.BlockSpec((1, 3, NB), lambda b, i: (b, 0, i)),
        ],
        out_specs=pl.BlockSpec((1, NB, W), lambda b, i: (b, i, 0)),
        out_shape=jax.ShapeDtypeStruct((B, N, W), jnp.float32),
    )(features, xyzT)


TOT = B * NPOINT * NSAMPLE
NW = 32
CH = 128
B_PER_W = TOT // NW
NCHUNK = B_PER_W // CH


def _gather_sc_body(table_hbm, idx_hbm, out_hbm, idx_v, rows_v, sem0, sem1):
    wid = lax.axis_index("s") * 2 + lax.axis_index("c")
    cbase = wid * NCHUNK
    pltpu.sync_copy(idx_hbm.at[pl.ds(cbase, NCHUNK)], idx_v)
    sems = [sem0, sem1]

    def start(j, slot):
        return pltpu.async_copy(table_hbm.at[idx_v.at[j]], rows_v.at[slot], sems[slot])

    cp = start(0, 0)
    for j in range(NCHUNK):
        slot = j % 2
        cp.wait()
        if j + 1 < NCHUNK:
            nxt = start(j + 1, (j + 1) % 2)
        pltpu.sync_copy(rows_v.at[slot], out_hbm.at[pl.ds((cbase + j) * CH, CH)])
        if j + 1 < NCHUNK:
            cp = nxt


def _gather_rows(table, idx2d):
    k = functools.partial(
        pl.kernel,
        mesh=plsc.VectorSubcoreMesh(core_axis_name="c", subcore_axis_name="s"),
        compiler_params=pltpu.CompilerParams(use_tc_tiling_on_sc=False),
        out_type=jax.ShapeDtypeStruct((TOT, W), jnp.float32),
        scratch_types=[
            pltpu.VMEM((NCHUNK, CH), jnp.int32),
            pltpu.VMEM((2, CH, W), jnp.float32),
            pltpu.SemaphoreType.DMA,
            pltpu.SemaphoreType.DMA,
        ],
    )(_gather_sc_body)
    return k(table, idx2d)


MQF = 128


def _final_body(g_ref, newxT_ref, valid_ref, w0p_ref, b0_ref, w1_ref, b1_ref,
                w2_ref, b2_ref, out_ref):
    bf = jnp.bfloat16
    rows = MQF * NSAMPLE
    g = g_ref[0]

    def qrep(d):
        qd = newxT_ref[0, d, :].reshape(MQF, 1)
        return jnp.broadcast_to(qd[:, None, :], (MQF, NSAMPLE, 1)).reshape(rows, 1)

    col = lax.broadcasted_iota(jnp.int32, (rows, W), 1)
    qp = (jnp.where(col == 64, qrep(0), 0.0)
          + jnp.where(col == 65, qrep(1), 0.0)
          + jnp.where(col == 66, qrep(2), 0.0))
    nfb = (g - qp).astype(bf)
    h = jnp.maximum(
        lax.dot_general(nfb, w0p_ref[...].astype(bf), (((1,), (1,)), ((), ())),
                        preferred_element_type=jnp.float32) + b0_ref[...][None, :], 0.0)
    h = jnp.maximum(
        lax.dot_general(h.astype(bf), w1_ref[...].astype(bf), (((1,), (1,)), ((), ())),
                        preferred_element_type=jnp.float32) + b1_ref[...][None, :], 0.0)
    h = jnp.maximum(
        lax.dot_general(h.astype(bf), w2_ref[...].astype(bf), (((1,), (1,)), ((), ())),
                        preferred_element_type=jnp.float32) + b2_ref[...][None, :], 0.0)
    h = h.reshape(MQF, NSAMPLE, 128)
    pooled = jnp.max(h, axis=1)
    pooled = pooled * valid_ref[0, 0].reshape(MQF, 1)
    out_ref[0] = pooled.T


def _mlp_pool(g, newxT, valid, w0p, b0, w1, b1, w2, b2):
    return pl.pallas_call(
        _final_body,
        grid=(B, NPOINT // MQF),
        in_specs=[
            pl.BlockSpec((1, MQF * NSAMPLE, W), lambda b, i: (b, i, 0)),
            pl.BlockSpec((1, 3, MQF), lambda b, i: (b, 0, i)),
            pl.BlockSpec((1, 1, MQF), lambda b, i: (b, 0, i)),
            pl.BlockSpec((64, W), lambda b, i: (0, 0)),
            pl.BlockSpec((64,), lambda b, i: (0,)),
            pl.BlockSpec((64, 64), lambda b, i: (0, 0)),
            pl.BlockSpec((64,), lambda b, i: (0,)),
            pl.BlockSpec((128, 64), lambda b, i: (0, 0)),
            pl.BlockSpec((128,), lambda b, i: (0,)),
        ],
        out_specs=pl.BlockSpec((1, 128, MQF), lambda b, i: (b, 0, i)),
        out_shape=jax.ShapeDtypeStruct((B, 128, NPOINT), jnp.float32),
    )(g, newxT, valid, w0p, b0, w1, b1, w2, b2)


def kernel(xyz, features, w0, b0, w1, b1, w2, b2):
    xyzT = jnp.transpose(xyz, (0, 2, 1))
    sidx, nx0, nx1, nx2 = _fps(xyzT)
    newxT = jnp.stack([nx0, nx1, nx2], axis=1)
    new_xyz = jnp.transpose(newxT, (0, 2, 1))
    xyz3 = xyzT.reshape(B, 3, NC, NL)
    idxg, valid = _select(xyz3, newxT)
    idx_flat = idxg.reshape(TOT // CH, CH)
    table = _build_table(features, xyzT).reshape(B * N, W)
    g = _gather_rows(table, idx_flat)
    g = g.reshape(B, NPOINT * NSAMPLE, W)
    w0p = jnp.concatenate(
        [w0[:, 3:], w0[:, :3], jnp.zeros((64, W - 67), jnp.float32)], axis=1)
    pooled = _mlp_pool(g, newxT, valid, w0p, b0, w1, b1, w2, b2)
    return new_xyz, pooled
